# baseline (speedup 1.0000x reference)
import functools

import jax
import jax.numpy as jnp
from jax import lax
from jax.experimental import pallas as pl
from jax.experimental.pallas import tpu as pltpu
from jax.experimental.pallas import tpu_sc as plsc

NC, NS, L = 2, 16, 16
NW = NC * NS
E = 320000
N_NODES = 100000
D = 128
NQ = D // L

A_G = 128
A_PER_W = E // NW

G = 256
W = 4096
GW = W // G
EG = E // NS
NWIN = EG // W
TAILV = EG - NWIN * W
TGF = TAILV // G
TLANE = TAILV - TGF * G
ECG = (E // NC) // NS
NWINC = ECG // W
TAILC = ECG - NWINC * W
CGF = TAILC // G
CLANE = TAILC - CGF * G
DUMP = N_NODES
ROWS_T = N_NODES // NS
ZR = 625
PAD = W

C_BLK = 4000


def _edge_products(n_fea, src, dst):
    mesh = plsc.VectorSubcoreMesh(core_axis_name="c", subcore_axis_name="s")

    nchunk = A_PER_W // A_G + 1

    @functools.partial(
        pl.kernel,
        mesh=mesh,
        out_type=jax.ShapeDtypeStruct((E, D), jnp.float32),
        scratch_types=[
            pltpu.VMEM((A_G,), jnp.int32),
            pltpu.VMEM((A_G,), jnp.int32),
            pltpu.VMEM((A_G,), jnp.int32),
            pltpu.VMEM((A_G,), jnp.int32),
            pltpu.VMEM((A_G, D), jnp.float32),
            pltpu.VMEM((A_G, D), jnp.float32),
            pltpu.VMEM((A_G, D), jnp.float32),
            pltpu.VMEM((A_G, D), jnp.float32),
            pltpu.SemaphoreType.DMA,
            pltpu.SemaphoreType.DMA,
            pltpu.SemaphoreType.DMA,
            pltpu.SemaphoreType.DMA,
        ],
    )
    def k(nfea_hbm, src_hbm, dst_hbm, x_hbm,
          si0, si1, di0, di1, rs0, rs1, rd0, rd1, sa0, sa1, sb0, sb1):
        wid = lax.axis_index("s") * NC + lax.axis_index("c")
        base0 = wid * A_PER_W
        si = (si0, si1)
        di = (di0, di1)
        rs = (rs0, rs1)
        rd = (rd0, rd1)
        sa = (sa0, sa1)
        sb = (sb0, sb1)

        def cbase(g):
            return base0 + jnp.minimum(g * A_G, A_PER_W - A_G)

        def issue(g, par):
            base = cbase(g)
            pltpu.sync_copy(src_hbm.at[pl.ds(base, A_G)], si[par])
            pltpu.sync_copy(dst_hbm.at[pl.ds(base, A_G)], di[par])
            pltpu.async_copy(nfea_hbm.at[si[par]], rs[par], sa[par])
            pltpu.async_copy(nfea_hbm.at[di[par]], rd[par], sb[par])

        def finish(g, par):
            pltpu.make_async_copy(nfea_hbm.at[si[par]], rs[par],
                                  sa[par]).wait()
            pltpu.make_async_copy(nfea_hbm.at[di[par]], rd[par],
                                  sb[par]).wait()

            def row(r, _):
                for cc in range(NQ):
                    sl = pl.ds(cc * L, L)
                    rs[par][r, sl] = rs[par][r, sl] * rd[par][r, sl]
                return 0

            lax.fori_loop(0, A_G, row, 0)
            pltpu.sync_copy(rs[par], x_hbm.at[pl.ds(cbase(g), A_G)])

        issue(0, 0)

        def pair(i, _):
            g0 = 2 * i
            issue(g0 + 1, 1)
            finish(g0, 0)
            issue(g0 + 2, 0)
            finish(g0 + 1, 1)
            return 0

        lax.fori_loop(0, (nchunk - 1) // 2, pair, 0)
        finish(nchunk - 1, 0)

    return k(n_fea, src, dst)


def _segment_sums(x16, srcp, dstp):
    mesh = plsc.VectorSubcoreMesh(core_axis_name="c", subcore_axis_name="s")

    @functools.partial(
        pl.kernel,
        mesh=mesh,
        out_type=(
            jax.ShapeDtypeStruct((N_NODES, D), jnp.float32),
            jax.ShapeDtypeStruct((NC, N_NODES, L), jnp.float32),
        ),
        compiler_params=pltpu.CompilerParams(use_tc_tiling_on_sc=False),
        scratch_types=[
            pltpu.VMEM((W,), jnp.int32),
            pltpu.VMEM((W,), jnp.int32),
            pltpu.VMEM((G, L), jnp.float32),
            pltpu.VMEM((G, L), jnp.float32),
            pltpu.VMEM((ZR, L), jnp.float32),
            pltpu.VMEM_SHARED((N_NODES + 8, L), jnp.float32),
            pltpu.SemaphoreType.DMA,
            pltpu.SemaphoreType.DMA,
            pltpu.SemaphoreType.DMA,
            pltpu.SemaphoreType.DMA,
        ],
    )
    def k(x16_hbm, src_hbm, dst_hbm, agg_hbm, cnt_hbm,
          gix1d, sid1d, grows0, grows1, zbuf, acc, sem0, sem1, sem2, sem3):
        c = lax.axis_index("c")
        s = lax.axis_index("s")

        def fill_z(r, _):
            zbuf[r, pl.ds(0, L)] = jnp.zeros((L,), jnp.float32)
            return 0

        lax.fori_loop(0, ZR, fill_z, 0)

        def zero_acc():
            for z in range(ROWS_T // ZR):
                pltpu.sync_copy(zbuf, acc.at[pl.ds(s * ROWS_T + z * ZR, ZR)])

        def copy_out(dst_ref):
            pltpu.sync_copy(acc.at[pl.ds(s * ROWS_T, ROWS_T)],
                            dst_ref.at[pl.ds(s * ROWS_T, ROWS_T)])

        def copy_out_cols(q):
            pltpu.sync_copy(acc.at[pl.ds(s * ROWS_T, ROWS_T)],
                            agg_hbm.at[pl.ds(s * ROWS_T, ROWS_T),
                                       pl.ds(q * L, L)])

        def window(base, gcnt, pad_from, q):
            pltpu.sync_copy(dst_hbm.at[pl.ds(base, W)], sid1d)
            for t in range(pad_from // L, G // L):
                sid1d[pl.ds((gcnt - 1) * G + t * L, L)] = jnp.full(
                    (L,), DUMP, jnp.int32)

            def sid(g):
                return sid1d.at[pl.ds(g * G, G)]

            if q is not None:
                pltpu.sync_copy(src_hbm.at[pl.ds(base, W)], gix1d)

                def fg(t, _):
                    sl = pl.ds(t * L, L)
                    gix1d[sl] = gix1d[sl] * NQ + q
                    return 0

                lax.fori_loop(0, gcnt * (G // L), fg, 0)

                bufs = (grows0, grows1)
                gsem = (sem0, sem1)
                ssem = (sem2, sem3)
                pend = pltpu.async_copy(
                    x16_hbm.at[gix1d.at[pl.ds(0, G)]], bufs[0], gsem[0])
                for g in range(gcnt):
                    nxt = None
                    if g + 1 < gcnt:
                        if g >= 1:
                            pltpu.make_async_copy(
                                bufs[(g - 1) % 2], acc.at[sid(g - 1)],
                                ssem[(g - 1) % 2]).wait()
                        nxt = pltpu.async_copy(
                            x16_hbm.at[gix1d.at[pl.ds((g + 1) * G, G)]],
                            bufs[(g + 1) % 2], gsem[(g + 1) % 2])
                    pend.wait()
                    pltpu.async_copy(bufs[g % 2], acc.at[sid(g)],
                                     ssem[g % 2], add=True)
                    pend = nxt
                for g in range(max(gcnt - 2, 0), gcnt):
                    pltpu.make_async_copy(
                        bufs[g % 2], acc.at[sid(g)], ssem[g % 2]).wait()
            else:
                for g in range(gcnt):
                    pltpu.async_copy(grows0, acc.at[sid(g)], sem0, add=True)
                for g in range(gcnt):
                    pltpu.make_async_copy(
                        grows0, acc.at[sid(g)], sem0).wait()

        for p in range(NQ // NC):
            q = c * (NQ // NC) + p
            zero_acc()
            plsc.subcore_barrier()

            def win(wi, _):
                window(s * EG + wi * W, GW, G, q)
                return 0

            lax.fori_loop(0, NWIN, win, 0)
            window(s * EG + NWIN * W, TGF + 1, TLANE, q)
            plsc.subcore_barrier()
            copy_out_cols(q)
            plsc.subcore_barrier()

        zero_acc()

        def fill_ones(r, _):
            grows0[r, pl.ds(0, L)] = jnp.ones((L,), jnp.float32)
            return 0

        lax.fori_loop(0, G, fill_ones, 0)
        plsc.subcore_barrier()
        cbase = c * (E // NC) + s * ECG

        def winc(wi, _):
            window(cbase + wi * W, GW, G, None)
            return 0

        lax.fori_loop(0, NWINC, winc, 0)
        window(cbase + NWINC * W, CGF + 1, CLANE, None)
        plsc.subcore_barrier()
        copy_out(cnt_hbm.at[c])

    return k(x16, srcp, dstp)


def _mlp_tail(h, w1_ref, b1_ref, w2_ref, b2_ref, w3_ref, b3_ref, out_ref):
    h = jnp.maximum(h, 0.0)
    h = jnp.maximum(
        jnp.dot(h, w1_ref[...], preferred_element_type=jnp.float32)
        + b1_ref[...], 0.0)
    h = jnp.maximum(
        jnp.dot(h, w2_ref[...], preferred_element_type=jnp.float32)
        + b2_ref[...], 0.0)
    lg = jnp.dot(h, w3_ref[...], preferred_element_type=jnp.float32)
    lg = lg + b3_ref[...]
    m = jnp.max(lg, axis=-1, keepdims=True)
    lse = m + jnp.log(jnp.sum(jnp.exp(lg - m), axis=-1, keepdims=True))
    out_ref[...] = lg - lse


def _full_spec(a):
    return pl.BlockSpec(a.shape, lambda i: (0,) * a.ndim)


def _dense_head_live(x, agg, cnt16, W_l, W_r, b_conv, W1, b1, W2, b2, W3, b3):
    def body(x_ref, agg_ref, cnt_ref, wl_ref, wr_ref, bc_ref,
             w1_ref, b1_ref, w2_ref, b2_ref, w3_ref, b3_ref, out_ref):
        base = jnp.dot(x_ref[...], wr_ref[...],
                       preferred_element_type=jnp.float32) + bc_ref[...]
        cnt = cnt_ref[0, :, 0:1] + cnt_ref[1, :, 0:1]
        mean = agg_ref[...] * (1.0 / jnp.maximum(cnt, 1.0))
        h = base + jnp.dot(mean, wl_ref[...],
                           preferred_element_type=jnp.float32)
        _mlp_tail(h, w1_ref, b1_ref, w2_ref, b2_ref, w3_ref, b3_ref, out_ref)

    return pl.pallas_call(
        body,
        grid=(N_NODES // C_BLK,),
        in_specs=[
            pl.BlockSpec((C_BLK, D), lambda i: (i, 0)),
            pl.BlockSpec((C_BLK, D), lambda i: (i, 0)),
            pl.BlockSpec((NC, C_BLK, L), lambda i: (0, i, 0)),
            _full_spec(W_l), _full_spec(W_r), _full_spec(b_conv),
            _full_spec(W1), _full_spec(b1), _full_spec(W2), _full_spec(b2),
            _full_spec(W3), _full_spec(b3),
        ],
        out_specs=pl.BlockSpec((C_BLK, 2), lambda i: (i, 0)),
        out_shape=jax.ShapeDtypeStruct((N_NODES, 2), jnp.float32),
    )(x, agg, cnt16, W_l, W_r, b_conv, W1, b1, W2, b2, W3, b3)


def _dense_head_rest(x, W_r, b_conv, W1, b1, W2, b2, W3, b3):
    def body(x_ref, wr_ref, bc_ref,
             w1_ref, b1_ref, w2_ref, b2_ref, w3_ref, b3_ref, out_ref):
        h = jnp.dot(x_ref[...], wr_ref[...],
                    preferred_element_type=jnp.float32) + bc_ref[...]
        _mlp_tail(h, w1_ref, b1_ref, w2_ref, b2_ref, w3_ref, b3_ref, out_ref)

    nrest = E - N_NODES
    off = N_NODES // C_BLK
    return pl.pallas_call(
        body,
        grid=(nrest // C_BLK,),
        in_specs=[
            pl.BlockSpec((C_BLK, D), lambda i: (i + off, 0)),
            _full_spec(W_r), _full_spec(b_conv),
            _full_spec(W1), _full_spec(b1), _full_spec(W2), _full_spec(b2),
            _full_spec(W3), _full_spec(b3),
        ],
        out_specs=pl.BlockSpec((C_BLK, 2), lambda i: (i, 0)),
        out_shape=jax.ShapeDtypeStruct((nrest, 2), jnp.float32),
    )(x, W_r, b_conv, W1, b1, W2, b2, W3, b3)


def kernel(drug_hidden_out, protein_hidden_out, all_edges,
           W_l, W_r, b_conv, W1, b1, W2, b2, W3, b3):
    n_fea = jnp.concatenate([drug_hidden_out, protein_hidden_out], axis=0)
    num_nodes = n_fea.shape[0]
    edges = all_edges[::2].T.astype(jnp.int32)
    num_pos = edges.shape[1]

    half = num_pos // 2
    k1, k2 = jax.random.split(jax.random.key(12345))
    ns_ = jax.random.randint(k1, (half,), 0, num_nodes, dtype=jnp.int32)
    nt_ = jax.random.randint(k2, (half,), 0, num_nodes, dtype=jnp.int32)
    src = jnp.concatenate([edges[0], ns_, nt_])
    dst = jnp.concatenate([edges[1], nt_, ns_])
    srcp = jnp.pad(src, (0, PAD))
    dstp = jnp.pad(dst, (0, PAD))

    x = _edge_products(n_fea, src, dst)
    x16 = x.reshape(E * NQ, L)
    bc, bb1, bb2, bb3 = (b_conv.reshape(1, D), b1.reshape(1, 64),
                         b2.reshape(1, 32), b3.reshape(1, 2))
    prob_rest = _dense_head_rest(x, W_r, bc, W1, bb1, W2, bb2, W3, bb3)
    agg, cnt16 = _segment_sums(x16, srcp, dstp)
    prob_live = _dense_head_live(x, agg, cnt16, W_l, W_r, bc,
                                 W1, bb1, W2, bb2, W3, bb3)
    prob = jnp.concatenate([prob_live, prob_rest], axis=0)
    label = jnp.concatenate(
        [jnp.ones((num_pos, 2), jnp.float32),
         jnp.zeros((num_pos, 2), jnp.float32)], axis=0)
    return (prob, label)


# speedup vs baseline: 1.0236x; 1.0236x over previous
import functools

import jax
import jax.numpy as jnp
from jax import lax
from jax.experimental import pallas as pl
from jax.experimental.pallas import tpu as pltpu
from jax.experimental.pallas import tpu_sc as plsc

NC, NS, L = 2, 16, 16
NW = NC * NS
E = 320000
N_NODES = 100000
D = 128
NQ = D // L

A_G = 128
A_PER_W = E // NW

G = 512
W = 2048
GW = W // G
EG = E // NS
NWIN = EG // W
TAILV = EG - NWIN * W
TGF = TAILV // G
TLANE = TAILV - TGF * G
ECG = (E // NC) // NS
NWINC = ECG // W
TAILC = ECG - NWINC * W
CGF = TAILC // G
CLANE = TAILC - CGF * G
DUMP = N_NODES
ROWS_T = N_NODES // NS
ZR = 625
PAD = W

C_BLK = 4000


def _edge_products(n_fea, src, dst):
    mesh = plsc.VectorSubcoreMesh(core_axis_name="c", subcore_axis_name="s")

    nchunk = A_PER_W // A_G + 1

    @functools.partial(
        pl.kernel,
        mesh=mesh,
        out_type=jax.ShapeDtypeStruct((E, D), jnp.float32),
        scratch_types=[
            pltpu.VMEM((A_G,), jnp.int32),
            pltpu.VMEM((A_G,), jnp.int32),
            pltpu.VMEM((A_G,), jnp.int32),
            pltpu.VMEM((A_G,), jnp.int32),
            pltpu.VMEM((A_G, D), jnp.float32),
            pltpu.VMEM((A_G, D), jnp.float32),
            pltpu.VMEM((A_G, D), jnp.float32),
            pltpu.VMEM((A_G, D), jnp.float32),
            pltpu.SemaphoreType.DMA,
            pltpu.SemaphoreType.DMA,
            pltpu.SemaphoreType.DMA,
            pltpu.SemaphoreType.DMA,
        ],
    )
    def k(nfea_hbm, src_hbm, dst_hbm, x_hbm,
          si0, si1, di0, di1, rs0, rs1, rd0, rd1, sa0, sa1, sb0, sb1):
        wid = lax.axis_index("s") * NC + lax.axis_index("c")
        base0 = wid * A_PER_W
        si = (si0, si1)
        di = (di0, di1)
        rs = (rs0, rs1)
        rd = (rd0, rd1)
        sa = (sa0, sa1)
        sb = (sb0, sb1)

        def cbase(g):
            return base0 + jnp.minimum(g * A_G, A_PER_W - A_G)

        def issue(g, par):
            base = cbase(g)
            pltpu.sync_copy(src_hbm.at[pl.ds(base, A_G)], si[par])
            pltpu.sync_copy(dst_hbm.at[pl.ds(base, A_G)], di[par])
            pltpu.async_copy(nfea_hbm.at[si[par]], rs[par], sa[par])
            pltpu.async_copy(nfea_hbm.at[di[par]], rd[par], sb[par])

        def finish(g, par):
            pltpu.make_async_copy(nfea_hbm.at[si[par]], rs[par],
                                  sa[par]).wait()
            pltpu.make_async_copy(nfea_hbm.at[di[par]], rd[par],
                                  sb[par]).wait()

            def row(r, _):
                for cc in range(NQ):
                    sl = pl.ds(cc * L, L)
                    rs[par][r, sl] = rs[par][r, sl] * rd[par][r, sl]
                return 0

            lax.fori_loop(0, A_G, row, 0)
            pltpu.sync_copy(rs[par], x_hbm.at[pl.ds(cbase(g), A_G)])

        issue(0, 0)

        def pair(i, _):
            g0 = 2 * i
            issue(g0 + 1, 1)
            finish(g0, 0)
            issue(g0 + 2, 0)
            finish(g0 + 1, 1)
            return 0

        lax.fori_loop(0, (nchunk - 1) // 2, pair, 0)
        finish(nchunk - 1, 0)

    return k(n_fea, src, dst)


def _segment_sums(x16, srcp, dstp):
    mesh = plsc.VectorSubcoreMesh(core_axis_name="c", subcore_axis_name="s")

    @functools.partial(
        pl.kernel,
        mesh=mesh,
        out_type=(
            jax.ShapeDtypeStruct((N_NODES, D), jnp.float32),
            jax.ShapeDtypeStruct((NC, N_NODES, L), jnp.float32),
        ),
        compiler_params=pltpu.CompilerParams(use_tc_tiling_on_sc=False),
        scratch_types=[
            pltpu.VMEM((W,), jnp.int32),
            pltpu.VMEM((W,), jnp.int32),
            pltpu.VMEM((G, L), jnp.float32),
            pltpu.VMEM((G, L), jnp.float32),
            pltpu.VMEM((ZR, L), jnp.float32),
            pltpu.VMEM_SHARED((N_NODES + 8, L), jnp.float32),
            pltpu.SemaphoreType.DMA,
            pltpu.SemaphoreType.DMA,
            pltpu.SemaphoreType.DMA,
            pltpu.SemaphoreType.DMA,
        ],
    )
    def k(x16_hbm, src_hbm, dst_hbm, agg_hbm, cnt_hbm,
          gix1d, sid1d, grows0, grows1, zbuf, acc, sem0, sem1, sem2, sem3):
        c = lax.axis_index("c")
        s = lax.axis_index("s")

        def fill_z(r, _):
            zbuf[r, pl.ds(0, L)] = jnp.zeros((L,), jnp.float32)
            return 0

        lax.fori_loop(0, ZR, fill_z, 0)

        def zero_acc():
            for z in range(ROWS_T // ZR):
                pltpu.sync_copy(zbuf, acc.at[pl.ds(s * ROWS_T + z * ZR, ZR)])

        def copy_out(dst_ref):
            pltpu.sync_copy(acc.at[pl.ds(s * ROWS_T, ROWS_T)],
                            dst_ref.at[pl.ds(s * ROWS_T, ROWS_T)])

        def copy_out_cols(q):
            pltpu.sync_copy(acc.at[pl.ds(s * ROWS_T, ROWS_T)],
                            agg_hbm.at[pl.ds(s * ROWS_T, ROWS_T),
                                       pl.ds(q * L, L)])

        def window(base, gcnt, pad_from, q):
            pltpu.sync_copy(dst_hbm.at[pl.ds(base, W)], sid1d)
            for t in range(pad_from // L, G // L):
                sid1d[pl.ds((gcnt - 1) * G + t * L, L)] = jnp.full(
                    (L,), DUMP, jnp.int32)

            def sid(g):
                return sid1d.at[pl.ds(g * G, G)]

            if q is not None:
                pltpu.sync_copy(src_hbm.at[pl.ds(base, W)], gix1d)

                def fg(t, _):
                    for u in range(4):
                        sl = pl.ds((t * 4 + u) * L, L)
                        gix1d[sl] = gix1d[sl] * NQ + q
                    return 0

                lax.fori_loop(0, gcnt * (G // L) // 4, fg, 0)

                bufs = (grows0, grows1)
                gsem = (sem0, sem1)
                ssem = (sem2, sem3)
                pend = pltpu.async_copy(
                    x16_hbm.at[gix1d.at[pl.ds(0, G)]], bufs[0], gsem[0])
                for g in range(gcnt):
                    nxt = None
                    if g + 1 < gcnt:
                        if g >= 1:
                            pltpu.make_async_copy(
                                bufs[(g - 1) % 2], acc.at[sid(g - 1)],
                                ssem[(g - 1) % 2]).wait()
                        nxt = pltpu.async_copy(
                            x16_hbm.at[gix1d.at[pl.ds((g + 1) * G, G)]],
                            bufs[(g + 1) % 2], gsem[(g + 1) % 2])
                    pend.wait()
                    pltpu.async_copy(bufs[g % 2], acc.at[sid(g)],
                                     ssem[g % 2], add=True)
                    pend = nxt
                for g in range(max(gcnt - 2, 0), gcnt):
                    pltpu.make_async_copy(
                        bufs[g % 2], acc.at[sid(g)], ssem[g % 2]).wait()
            else:
                for g in range(gcnt):
                    pltpu.async_copy(grows0, acc.at[sid(g)], sem0, add=True)
                for g in range(gcnt):
                    pltpu.make_async_copy(
                        grows0, acc.at[sid(g)], sem0).wait()

        for p in range(NQ // NC):
            q = c * (NQ // NC) + p
            zero_acc()
            plsc.subcore_barrier()

            def win(wi, _):
                window(s * EG + wi * W, GW, G, q)
                return 0

            lax.fori_loop(0, NWIN, win, 0)
            window(s * EG + NWIN * W, TGF + 1, TLANE, q)
            plsc.subcore_barrier()
            copy_out_cols(q)
            plsc.subcore_barrier()

        zero_acc()

        def fill_ones(r, _):
            grows0[r, pl.ds(0, L)] = jnp.ones((L,), jnp.float32)
            return 0

        lax.fori_loop(0, G, fill_ones, 0)
        plsc.subcore_barrier()
        cbase = c * (E // NC) + s * ECG

        def winc(wi, _):
            window(cbase + wi * W, GW, G, None)
            return 0

        lax.fori_loop(0, NWINC, winc, 0)
        window(cbase + NWINC * W, CGF + 1, CLANE, None)
        plsc.subcore_barrier()
        copy_out(cnt_hbm.at[c])

    return k(x16, srcp, dstp)


def _mlp_tail(h, w1_ref, b1_ref, w2_ref, b2_ref, w3_ref, b3_ref, out_ref):
    h = jnp.maximum(h, 0.0)
    h = jnp.maximum(
        jnp.dot(h, w1_ref[...], preferred_element_type=jnp.float32)
        + b1_ref[...], 0.0)
    h = jnp.maximum(
        jnp.dot(h, w2_ref[...], preferred_element_type=jnp.float32)
        + b2_ref[...], 0.0)
    lg = jnp.dot(h, w3_ref[...], preferred_element_type=jnp.float32)
    lg = lg + b3_ref[...]
    m = jnp.max(lg, axis=-1, keepdims=True)
    lse = m + jnp.log(jnp.sum(jnp.exp(lg - m), axis=-1, keepdims=True))
    out_ref[...] = lg - lse


def _full_spec(a):
    return pl.BlockSpec(a.shape, lambda i: (0,) * a.ndim)


def _dense_head_live(x, agg, cnt16, W_l, W_r, b_conv, W1, b1, W2, b2, W3, b3):
    def body(x_ref, agg_ref, cnt_ref, wl_ref, wr_ref, bc_ref,
             w1_ref, b1_ref, w2_ref, b2_ref, w3_ref, b3_ref, out_ref):
        base = jnp.dot(x_ref[...], wr_ref[...], precision=lax.Precision.DEFAULT,
                       preferred_element_type=jnp.float32) + bc_ref[...]
        cnt = cnt_ref[0, :, 0:1] + cnt_ref[1, :, 0:1]
        mean = agg_ref[...] * (1.0 / jnp.maximum(cnt, 1.0))
        h = base + jnp.dot(mean, wl_ref[...], precision=lax.Precision.DEFAULT,
                           preferred_element_type=jnp.float32)
        _mlp_tail(h, w1_ref, b1_ref, w2_ref, b2_ref, w3_ref, b3_ref, out_ref)

    return pl.pallas_call(
        body,
        grid=(N_NODES // C_BLK,),
        in_specs=[
            pl.BlockSpec((C_BLK, D), lambda i: (i, 0)),
            pl.BlockSpec((C_BLK, D), lambda i: (i, 0)),
            pl.BlockSpec((NC, C_BLK, L), lambda i: (0, i, 0)),
            _full_spec(W_l), _full_spec(W_r), _full_spec(b_conv),
            _full_spec(W1), _full_spec(b1), _full_spec(W2), _full_spec(b2),
            _full_spec(W3), _full_spec(b3),
        ],
        out_specs=pl.BlockSpec((C_BLK, 2), lambda i: (i, 0)),
        out_shape=jax.ShapeDtypeStruct((N_NODES, 2), jnp.float32),
    )(x, agg, cnt16, W_l, W_r, b_conv, W1, b1, W2, b2, W3, b3)


def _dense_head_rest(x, W_r, b_conv, W1, b1, W2, b2, W3, b3):
    def body(x_ref, wr_ref, bc_ref,
             w1_ref, b1_ref, w2_ref, b2_ref, w3_ref, b3_ref, out_ref):
        h = jnp.dot(x_ref[...], wr_ref[...], precision=lax.Precision.DEFAULT,
                    preferred_element_type=jnp.float32) + bc_ref[...]
        _mlp_tail(h, w1_ref, b1_ref, w2_ref, b2_ref, w3_ref, b3_ref, out_ref)

    nrest = E - N_NODES
    off = N_NODES // C_BLK
    return pl.pallas_call(
        body,
        grid=(nrest // C_BLK,),
        in_specs=[
            pl.BlockSpec((C_BLK, D), lambda i: (i + off, 0)),
            _full_spec(W_r), _full_spec(b_conv),
            _full_spec(W1), _full_spec(b1), _full_spec(W2), _full_spec(b2),
            _full_spec(W3), _full_spec(b3),
        ],
        out_specs=pl.BlockSpec((C_BLK, 2), lambda i: (i, 0)),
        out_shape=jax.ShapeDtypeStruct((nrest, 2), jnp.float32),
    )(x, W_r, b_conv, W1, b1, W2, b2, W3, b3)


def kernel(drug_hidden_out, protein_hidden_out, all_edges,
           W_l, W_r, b_conv, W1, b1, W2, b2, W3, b3):
    n_fea = jnp.concatenate([drug_hidden_out, protein_hidden_out], axis=0)
    num_nodes = n_fea.shape[0]
    edges = all_edges[::2].T.astype(jnp.int32)
    num_pos = edges.shape[1]

    half = num_pos // 2
    k1, k2 = jax.random.split(jax.random.key(12345))
    ns_ = jax.random.randint(k1, (half,), 0, num_nodes, dtype=jnp.int32)
    nt_ = jax.random.randint(k2, (half,), 0, num_nodes, dtype=jnp.int32)
    src = jnp.concatenate([edges[0], ns_, nt_])
    dst = jnp.concatenate([edges[1], nt_, ns_])
    srcp = jnp.pad(src, (0, PAD))
    dstp = jnp.pad(dst, (0, PAD))

    x = _edge_products(n_fea, src, dst)
    x16 = x.reshape(E * NQ, L)
    bc, bb1, bb2, bb3 = (b_conv.reshape(1, D), b1.reshape(1, 64),
                         b2.reshape(1, 32), b3.reshape(1, 2))
    prob_rest = _dense_head_rest(x, W_r, bc, W1, bb1, W2, bb2, W3, bb3)
    agg, cnt16 = _segment_sums(x16, srcp, dstp)
    prob_live = _dense_head_live(x, agg, cnt16, W_l, W_r, bc,
                                 W1, bb1, W2, bb2, W3, bb3)
    prob = jnp.concatenate([prob_live, prob_rest], axis=0)
    label = jnp.concatenate(
        [jnp.ones((num_pos, 2), jnp.float32),
         jnp.zeros((num_pos, 2), jnp.float32)], axis=0)
    return (prob, label)
